# baseline (speedup 1.0000x reference)
import functools

import jax
import jax.numpy as jnp
from jax import lax
from jax.experimental import pallas as pl
from jax.experimental.pallas import tpu as pltpu
from jax.experimental.pallas import tpu_sc as plsc

_N = 4096
_F = 40
_V = 100000
_D = 16
_FD = 13
_FS = 27
_P = _N * _FS
_NC = 2
_NSUB = 16
_NW = _NC * _NSUB
_CHUNK = _P // _NW
_JR = _CHUNK // 128

_TN = 512
_GRID = _N // _TN
_CD = _FD * _D
_CS = _FS * _D


def _sc_body(idx_hbm, off_hbm, t1_hbm, t2_hbm, out1_hbm, out2_hbm,
             idx_v, off_v, rows1_v, rows2_v, sem1, sem2):
    cid = lax.axis_index("c")
    sid = lax.axis_index("s")
    wid = sid * _NC + cid
    pbase = wid * _CHUNK

    pltpu.sync_copy(idx_hbm.at[pl.ds(pbase, _CHUNK)], idx_v)
    pltpu.sync_copy(off_hbm, off_v)

    def _add(i, carry):
        c = i * 16
        idx_v[pl.ds(c, 16)] = idx_v[pl.ds(c, 16)] + off_v[pl.ds(c, 16)]
        return carry

    lax.fori_loop(0, _CHUNK // 16, _add, 0)

    def _fire(j, carry):
        ids = idx_v.at[pl.ds(j * 128, 128)]
        pltpu.make_async_copy(
            t1_hbm.at[ids], rows1_v.at[pl.ds(j * 128, 128)], sem1).start()
        pltpu.make_async_copy(
            t2_hbm.at[ids], rows2_v.at[pl.ds(j * 128, 128)], sem2).start()
        return carry

    lax.fori_loop(0, _JR, _fire, 0)

    def _drain(j, carry):
        ids = idx_v.at[pl.ds(j * 128, 128)]
        pltpu.make_async_copy(
            t1_hbm.at[ids], rows1_v.at[pl.ds(j * 128, 128)], sem1).wait()
        pltpu.make_async_copy(
            t2_hbm.at[ids], rows2_v.at[pl.ds(j * 128, 128)], sem2).wait()
        return carry

    lax.fori_loop(0, _JR, _drain, 0)

    pltpu.sync_copy(rows1_v, out1_hbm.at[pl.ds(pbase, _CHUNK)])
    pltpu.sync_copy(rows2_v, out2_hbm.at[pl.ds(pbase, _CHUNK)])


def _sc_gather(idx2d, offs, t1f, t2f):
    mesh = plsc.VectorSubcoreMesh(core_axis_name="c", subcore_axis_name="s")
    f = pl.kernel(
        _sc_body,
        out_type=(
            jax.ShapeDtypeStruct((_P, _D), jnp.float32),
            jax.ShapeDtypeStruct((_P, _D), jnp.float32),
        ),
        mesh=mesh,
        scratch_types=[
            pltpu.VMEM((_CHUNK,), jnp.int32),
            pltpu.VMEM((_CHUNK,), jnp.int32),
            pltpu.VMEM((_CHUNK, _D), jnp.float32),
            pltpu.VMEM((_CHUNK, _D), jnp.float32),
            pltpu.SemaphoreType.DMA,
            pltpu.SemaphoreType.DMA,
        ],
        compiler_params=pltpu.CompilerParams(use_tc_tiling_on_sc=False),
    )
    return f(idx2d, offs, t1f, t2f)


def _tc_body(xid_ref, xv13_ref, xv27_ref, g1_ref, g2_ref,
             w1f_ref, b1f_ref, w2f_ref, b2f_ref,
             w1td_ref, w1ts_ref, l1b_ref, bn1g_ref, bn1b_ref,
             w2t_ref, l2b_ref, bn2g_ref, bn2b_ref, bias_ref,
             out_ref, hbuf):
    i = pl.program_id(0)

    xid = xid_ref[...]
    xv13 = xv13_ref[...]
    xv27 = xv27_ref[...]
    g1 = g1_ref[...]
    g2 = g2_ref[...]

    c208 = lax.broadcasted_iota(jnp.int32, (_FD, _CD), 1)
    f208 = lax.broadcasted_iota(jnp.int32, (_FD, _CD), 0)
    r208 = (c208 // _D == f208).astype(jnp.float32)
    c432 = lax.broadcasted_iota(jnp.int32, (_FS, _CS), 1)
    f432 = lax.broadcasted_iota(jnp.int32, (_FS, _CS), 0)
    r432 = (c432 // _D == f432).astype(jnp.float32)
    cc208 = lax.broadcasted_iota(jnp.int32, (_CD, _D), 0)
    dd208 = lax.broadcasted_iota(jnp.int32, (_CD, _D), 1)
    s208 = (cc208 % _D == dd208).astype(jnp.float32)
    cc432 = lax.broadcasted_iota(jnp.int32, (_CS, _D), 0)
    dd432 = lax.broadcasted_iota(jnp.int32, (_CS, _D), 1)
    s432 = (cc432 % _D == dd432).astype(jnp.float32)

    dot = functools.partial(jnp.dot, preferred_element_type=jnp.float32)

    xidr = dot(xid, r208)
    xvrd = dot(xv13, r208)
    xvrs = dot(xv27, r432)

    e1d = (xidr * w1f_ref[...] + b1f_ref[...]) * xvrd
    e2d = (xidr * w2f_ref[...] + b2f_ref[...]) * xvrd
    e1s = g1 * xvrs
    e2s = g2 * xvrs

    fm1 = (jnp.sum(e1d, axis=1, keepdims=True)
           + jnp.sum(e1s, axis=1, keepdims=True))

    s = dot(e2d, s208) + dot(e2s, s432)
    sq = dot(e2d * e2d, s208) + dot(e2s * e2s, s432)
    sec = 0.5 * jnp.sum(s * s - sq, axis=1, keepdims=True)

    h = dot(e2d, w1td_ref[...]) + dot(e2s, w1ts_ref[...]) + l1b_ref[...]
    hbuf[pl.ds(i * _TN, _TN), :] = h

    out_ref[pl.ds(i * _TN, _TN), :] = fm1 + sec + bias_ref[...]

    @pl.when(i == _GRID - 1)
    def _finish():
        hall = hbuf[...]
        mu1 = jnp.mean(hall, axis=0, keepdims=True)
        var1 = jnp.mean((hall - mu1) * (hall - mu1), axis=0, keepdims=True)
        h1 = bn1g_ref[...] * (hall - mu1) / jnp.sqrt(var1 + 1e-5) + bn1b_ref[...]
        h2 = jnp.dot(h1, w2t_ref[...], preferred_element_type=jnp.float32) + l2b_ref[...]
        mu2 = jnp.mean(h2, axis=0, keepdims=True)
        var2 = jnp.mean((h2 - mu2) * (h2 - mu2), axis=0, keepdims=True)
        h2n = bn2g_ref[...] * (h2 - mu2) / jnp.sqrt(var2 + 1e-5) + bn2b_ref[...]
        out_ref[...] = out_ref[...] + jnp.sum(h2n, axis=1, keepdims=True)


def _tc_compute(xid, xv13, xv27, g1, g2, w1f, b1f, w2f, b2f,
                w1td, w1ts, l1b, bn1g, bn1b, w2t, l2b, bn2g, bn2b, bias2d):
    row_blk = lambda c: pl.BlockSpec((_TN, c), lambda i: (i, 0))
    full_blk = lambda r, c: pl.BlockSpec((r, c), lambda i: (0, 0))
    return pl.pallas_call(
        _tc_body,
        grid=(_GRID,),
        in_specs=[
            row_blk(_FD), row_blk(_FD), row_blk(_FS),
            row_blk(_CS), row_blk(_CS),
            full_blk(1, _CD), full_blk(1, _CD), full_blk(1, _CD), full_blk(1, _CD),
            full_blk(_CD, 32), full_blk(_CS, 32), full_blk(1, 32),
            full_blk(1, 32), full_blk(1, 32),
            full_blk(32, 32), full_blk(1, 32), full_blk(1, 32), full_blk(1, 32),
            row_blk(1),
        ],
        out_specs=pl.BlockSpec((_N, 1), lambda i: (0, 0)),
        out_shape=jax.ShapeDtypeStruct((_N, 1), jnp.float32),
        scratch_shapes=[pltpu.VMEM((_N, 32), jnp.float32)],
    )(xid, xv13, xv27, g1, g2, w1f, b1f, w2f, b2f,
      w1td, w1ts, l1b, bn1g, bn1b, w2t, l2b, bn2g, bn2b, bias2d)


def kernel(Xi, Xv, W1d, b1d, T1, W2d, b2d, T2, lin1_W, lin1_b,
           bn1_g, bn1_b, lin2_W, lin2_b, bn2_g, bn2_b, bias):
    xid = Xi[:, :_FD, 0].astype(jnp.float32)
    idx2d = Xi[:, _FD:, 0].astype(jnp.int32).reshape(_P)
    offs = (jnp.arange(_CHUNK, dtype=jnp.int32) % _FS) * _V
    t1f = T1.reshape(_FS * _V, _D)
    t2f = T2.reshape(_FS * _V, _D)

    g1, g2 = _sc_gather(idx2d, offs, t1f, t2f)
    g1 = g1.reshape(_N, _CS)
    g2 = g2.reshape(_N, _CS)

    xv13 = Xv[:, :_FD]
    xv27 = Xv[:, _FD:]
    w1f = W1d.reshape(1, _CD)
    b1f = b1d.reshape(1, _CD)
    w2f = W2d.reshape(1, _CD)
    b2f = b2d.reshape(1, _CD)
    w1td = lin1_W[:, :_CD].T
    w1ts = lin1_W[:, _CD:].T
    l1b = lin1_b.reshape(1, 32)
    w2t = lin2_W.T
    l2b = lin2_b.reshape(1, 32)
    out = _tc_compute(xid, xv13, xv27, g1, g2, w1f, b1f, w2f, b2f,
                      w1td, w1ts, l1b, bn1_g.reshape(1, 32), bn1_b.reshape(1, 32),
                      w2t, l2b, bn2_g.reshape(1, 32), bn2_b.reshape(1, 32),
                      bias.reshape(_N, 1))
    return out.reshape(_N)
